# baseline (speedup 1.0000x reference)
import jax
import jax.numpy as jnp
import numpy as np
from jax import lax
from jax.experimental import pallas as pl
from jax.experimental.pallas import tpu as pltpu

_EARTH_RADIUS = 6371.0
_SPAN = 1500.0
_THETA = _SPAN / _EARTH_RADIUS
_COS_THR = float(np.cos(_THETA))
_NEG = float(np.finfo(np.float32).min)

_BQ = 128
_GD = 16


def _flash_body(qg_ref, kvg_ref, q_ref, k_ref, v_ref, o_ref, bias_ref):
    h = pl.program_id(0)
    qi = pl.program_id(1)
    S = kvg_ref.shape[1]
    D = q_ref.shape[-1]

    @pl.when(jnp.logical_and(h == 0, qi == 0))
    def _():
        klat = kvg_ref[0:1, :]
        klon = kvg_ref[1:2, :]
        k_sl = jnp.sin(klat)
        k_cl = jnp.cos(klat)
        k_a = k_cl * jnp.sin(klon)
        k_b = k_cl * jnp.cos(klon)
        qlat = qg_ref[:, 0:1]
        qlon = qg_ref[:, 1:2]
        q_sl = jnp.sin(qlat)
        q_cl = jnp.cos(qlat)
        q_a = q_cl * jnp.sin(qlon)
        q_b = q_cl * jnp.cos(qlon)
        g = q_sl * k_sl + q_a * k_a + q_b * k_b
        bias_ref[...] = jnp.where(g >= _COS_THR, 0.0, _NEG)

    scale = float(1.0 / np.sqrt(D))
    qb = (q_ref[0, 0] * scale).astype(jnp.bfloat16)
    kb = k_ref[0, 0].astype(jnp.bfloat16)
    vb = v_ref[0, 0].astype(jnp.bfloat16)

    base = pl.multiple_of(qi * _BQ, _BQ)
    s = lax.dot_general(qb, kb, (((1,), (1,)), ((), ())),
                        preferred_element_type=jnp.float32)
    s = s + bias_ref[pl.ds(base, _BQ), :]
    m = jnp.max(s, axis=1, keepdims=True)
    p = jnp.exp(s - m)
    denom = jnp.sum(p, axis=1, keepdims=True)
    o = lax.dot_general(p.astype(jnp.bfloat16), vb,
                        (((1,), (0,)), ((), ())),
                        preferred_element_type=jnp.float32)
    o_ref[0, 0] = o / denom


def kernel(q, k, v, q_lat, q_lon, kv_lat, kv_lon):
    B, H, S, D = q.shape
    nq = S // _BQ

    pad = jnp.zeros((S, _GD - 2), jnp.float32)
    qg = jnp.concatenate([q_lat[:, None], q_lon[:, None], pad], axis=1)
    kvg = jnp.concatenate(
        [kv_lat[None, :], kv_lon[None, :],
         jnp.zeros((6, S), jnp.float32)], axis=0)

    grid = (H, nq)
    out = pl.pallas_call(
        _flash_body,
        grid=grid,
        in_specs=[
            pl.BlockSpec((S, _GD), lambda h, qi: (0, 0)),
            pl.BlockSpec((8, S), lambda h, qi: (0, 0)),
            pl.BlockSpec((1, 1, _BQ, D), lambda h, qi: (0, h, qi, 0)),
            pl.BlockSpec((1, 1, S, D), lambda h, qi: (0, h, 0, 0)),
            pl.BlockSpec((1, 1, S, D), lambda h, qi: (0, h, 0, 0)),
        ],
        out_specs=pl.BlockSpec((1, 1, _BQ, D), lambda h, qi: (0, h, qi, 0)),
        out_shape=jax.ShapeDtypeStruct((B, H, S, D), jnp.float32),
        scratch_shapes=[pltpu.VMEM((S, S), jnp.float32)],
    )(qg, kvg, q, k, v)
    return out


# speedup vs baseline: 1.2885x; 1.2885x over previous
import jax
import jax.numpy as jnp
import numpy as np
from jax import lax
from jax.experimental import pallas as pl
from jax.experimental.pallas import tpu as pltpu

_EARTH_RADIUS = 6371.0
_SPAN = 1500.0
_THETA = _SPAN / _EARTH_RADIUS
_COS_THR = float(np.cos(_THETA))
_NEG = float(np.finfo(np.float32).min)

_BQ = 256
_GD = 16


def _flash_body(qg_ref, kvg_ref, q_ref, k_ref, v_ref, o_ref, bias_ref):
    h = pl.program_id(0)
    qi = pl.program_id(1)
    S = kvg_ref.shape[1]
    D = q_ref.shape[-1]

    @pl.when(jnp.logical_and(h == 0, qi == 0))
    def _():
        klat = kvg_ref[0:1, :]
        klon = kvg_ref[1:2, :]
        k_sl = jnp.sin(klat)
        k_cl = jnp.cos(klat)
        k_a = k_cl * jnp.sin(klon)
        k_b = k_cl * jnp.cos(klon)
        qlat = qg_ref[:, 0:1]
        qlon = qg_ref[:, 1:2]
        q_sl = jnp.sin(qlat)
        q_cl = jnp.cos(qlat)
        q_a = q_cl * jnp.sin(qlon)
        q_b = q_cl * jnp.cos(qlon)
        g = q_sl * k_sl + q_a * k_a + q_b * k_b
        bias_ref[...] = jnp.where(g >= _COS_THR, 0.0, _NEG)

    scale = float(1.0 / np.sqrt(D))
    qb = (q_ref[0, 0] * scale).astype(jnp.bfloat16)
    kb = k_ref[0, 0].astype(jnp.bfloat16)
    vb = v_ref[0, 0].astype(jnp.bfloat16)

    base = pl.multiple_of(qi * _BQ, _BQ)
    s = lax.dot_general(qb, kb, (((1,), (1,)), ((), ())),
                        preferred_element_type=jnp.float32)
    s = s + bias_ref[pl.ds(base, _BQ), :]
    m = jnp.max(s, axis=1, keepdims=True)
    p = jnp.exp(s - m)
    denom = jnp.sum(p, axis=1, keepdims=True)
    o = lax.dot_general(p.astype(jnp.bfloat16), vb,
                        (((1,), (0,)), ((), ())),
                        preferred_element_type=jnp.float32)
    o_ref[0, 0] = o / denom


def kernel(q, k, v, q_lat, q_lon, kv_lat, kv_lon):
    B, H, S, D = q.shape
    nq = S // _BQ

    pad = jnp.zeros((S, _GD - 2), jnp.float32)
    qg = jnp.concatenate([q_lat[:, None], q_lon[:, None], pad], axis=1)
    kvg = jnp.concatenate(
        [kv_lat[None, :], kv_lon[None, :],
         jnp.zeros((6, S), jnp.float32)], axis=0)

    grid = (H, nq)
    out = pl.pallas_call(
        _flash_body,
        grid=grid,
        in_specs=[
            pl.BlockSpec((S, _GD), lambda h, qi: (0, 0)),
            pl.BlockSpec((8, S), lambda h, qi: (0, 0)),
            pl.BlockSpec((1, 1, _BQ, D), lambda h, qi: (0, h, qi, 0)),
            pl.BlockSpec((1, 1, S, D), lambda h, qi: (0, h, 0, 0)),
            pl.BlockSpec((1, 1, S, D), lambda h, qi: (0, h, 0, 0)),
        ],
        out_specs=pl.BlockSpec((1, 1, _BQ, D), lambda h, qi: (0, h, qi, 0)),
        out_shape=jax.ShapeDtypeStruct((B, H, S, D), jnp.float32),
        scratch_shapes=[pltpu.VMEM((S, S), jnp.float32)],
    )(qg, kvg, q, k, v)
    return out
